# Initial kernel scaffold; baseline (speedup 1.0000x reference)
import jax
import jax.numpy as jnp
from jax.experimental import pallas as pl


def kernel(x_atom, edge_index, edge_attr, node_mask, atom_emb, bond_emb, eps, W1, b1, W2, b2, pred_W, pred_b):
    raise NotImplementedError("write your pallas kernel here")


# speedup vs baseline: 2.8066x; 2.8066x over previous
import functools
import jax
import jax.numpy as jnp
from jax import lax
from jax.experimental import pallas as pl
from jax.experimental.pallas import tpu as pltpu
from jax.experimental.pallas import tpu_sc as plsc

NC, NS = 2, 16
EPAD, HALF = 320, 160
NVR = HALF // 16
NPAD = 10240
RPAD = 10368
CK = 128
BOND_DIMS = (5, 6, 2)


def _mesh():
    return plsc.VectorSubcoreMesh(core_axis_name="c", subcore_axis_name="s")


def _zero_rows(buf, nrows):
    z = jnp.zeros((16,), jnp.float32)
    def body(r, _):
        for k in range(NVR):
            buf[r, pl.ds(16 * k, 16)] = z
        return 0
    lax.fori_loop(0, nrows, body, 0)


def _atom_body(table_hbm, aidx_hbm, h_hbm, sem0, sem1):
    c = lax.axis_index("c")
    s = lax.axis_index("s")
    def scoped(idxv, gh):
        _atom_inner(table_hbm, aidx_hbm, h_hbm, sem0, sem1, c, s, idxv, gh)
    pl.run_scoped(scoped,
                  pltpu.VMEM((45, CK), jnp.int32),
                  pltpu.VMEM((CK, HALF), jnp.float32))


def _atom_inner(table_hbm, aidx_hbm, h_hbm, sem0, sem1, c, s, idxv, gh):
    pltpu.sync_copy(aidx_hbm.at[c].at[s], idxv)
    def chunk(j, _):
        pltpu.async_copy(table_hbm.at[idxv.at[j * 9]], gh, sem0).wait()
        def feat(f, _):
            pltpu.async_copy(table_hbm.at[idxv.at[j * 9 + f]], gh, sem1,
                             add=True).wait()
            return 0
        lax.fori_loop(1, 9, feat, 0)
        base = c * NPAD + s * 640 + j * CK
        pltpu.sync_copy(gh, h_hbm.at[pl.ds(base, CK)])
        return 0
    lax.fori_loop(0, 5, chunk, 0)


def _layer_body(h_hbm, bond_hbm, src_hbm, dst_hbm, combo_hbm, agg_hbm,
                aggs, sem0, sem1):
    c = lax.axis_index("c")
    s = lax.axis_index("s")
    def scoped(srcv, dstv, combov, gh):
        _layer_inner(h_hbm, bond_hbm, src_hbm, dst_hbm, combo_hbm, agg_hbm,
                     aggs, sem0, sem1, c, s, srcv, dstv, combov, gh)
    pl.run_scoped(scoped,
                  pltpu.VMEM((CK,), jnp.int32),
                  pltpu.VMEM((CK,), jnp.int32),
                  pltpu.VMEM((CK,), jnp.int32),
                  pltpu.VMEM((CK, HALF), jnp.float32))


def _layer_inner(h_hbm, bond_hbm, src_hbm, dst_hbm, combo_hbm, agg_hbm,
                 aggs, sem0, sem1, c, s, srcv, dstv, combov, gh):
    _zero_rows(gh, CK)
    def zchunk(i, _):
        idx = s + 16 * i
        @pl.when(idx < RPAD // CK)
        def _():
            pltpu.sync_copy(gh, aggs.at[pl.ds(idx * CK, CK)])
        return 0
    lax.fori_loop(0, 6, zchunk, 0)
    plsc.subcore_barrier()
    nchunk = src_hbm.shape[2]
    def chunk(j, _):
        pltpu.sync_copy(src_hbm.at[c].at[s].at[j], srcv)
        pltpu.sync_copy(combo_hbm.at[c].at[s].at[j], combov)
        pltpu.sync_copy(dst_hbm.at[s].at[j], dstv)
        pltpu.async_copy(h_hbm.at[srcv], gh, sem0).wait()
        pltpu.async_copy(bond_hbm.at[combov], gh, sem1, add=True).wait()
        def row(r, _):
            for k in range(NVR):
                sl = pl.ds(16 * k, 16)
                gh[r, sl] = jnp.maximum(gh[r, sl], 0.0)
            return 0
        lax.fori_loop(0, CK, row, 0)
        pltpu.sync_copy(gh, aggs.at[dstv], add=True)
        return 0
    lax.fori_loop(0, nchunk, chunk, 0)
    plsc.subcore_barrier()
    def ochunk(i, _):
        idx = s + 16 * i
        @pl.when(idx < RPAD // CK)
        def _():
            pltpu.sync_copy(aggs.at[pl.ds(idx * CK, CK)],
                            agg_hbm.at[pl.ds(c * RPAD + idx * CK, CK)])
        return 0
    lax.fori_loop(0, 6, ochunk, 0)


def _atom_encoder(table_flat, aidx2):
    k = pl.kernel(
        _atom_body,
        out_type=jax.ShapeDtypeStruct((NC * NPAD, HALF), jnp.float32),
        mesh=_mesh(),
        compiler_params=pltpu.CompilerParams(use_tc_tiling_on_sc=False),
        scratch_types=[
            pltpu.SemaphoreType.DMA,
            pltpu.SemaphoreType.DMA,
        ],
    )
    return k(table_flat, aidx2)


def _message_pass(h_flat, bond_flat, src2, dst3, combo2):
    nchunk = src2.shape[2]
    k = pl.kernel(
        _layer_body,
        out_type=jax.ShapeDtypeStruct((NC * RPAD, HALF), jnp.float32),
        mesh=_mesh(),
        compiler_params=pltpu.CompilerParams(use_tc_tiling_on_sc=False),
        scratch_types=[
            pltpu.VMEM_SHARED((RPAD, HALF), jnp.float32),
            pltpu.SemaphoreType.DMA,
            pltpu.SemaphoreType.DMA,
        ],
    )
    return k(h_flat, bond_flat, src2, dst3, combo2)


def _mlp_tc_body(scale_ref, h_ref, a_ref, w1_ref, b1_ref, w2_ref, b2_ref,
                 o_ref, *, last):
    x = jnp.concatenate([h_ref[0], h_ref[1]], axis=-1)
    a = jnp.concatenate([a_ref[0], a_ref[1]], axis=-1)
    z = x * scale_ref[0, 0] + a
    t = jnp.maximum(
        jax.lax.dot_general(z, w1_ref[...], (((1,), (0,)), ((), ())),
                            preferred_element_type=jnp.float32) + b1_ref[...],
        0.0)
    y = jax.lax.dot_general(t, w2_ref[...], (((1,), (0,)), ((), ())),
                            preferred_element_type=jnp.float32) + b2_ref[...]
    if not last:
        y = jnp.maximum(y, 0.0)
    o_ref[0] = y[:, :HALF]
    o_ref[1] = y[:, HALF:]


def _mlp_tc(h2, agg2, w1, b1, w2, b2, scale, last):
    bn = 1000
    grid = (10000 // bn,)
    return pl.pallas_call(
        functools.partial(_mlp_tc_body, last=last),
        grid=grid,
        in_specs=[
            pl.BlockSpec((1, 1), lambda i: (0, 0), memory_space=pltpu.SMEM),
            pl.BlockSpec((2, bn, HALF), lambda i: (0, i, 0)),
            pl.BlockSpec((2, bn, HALF), lambda i: (0, i, 0)),
            pl.BlockSpec((EPAD, 600), lambda i: (0, 0)),
            pl.BlockSpec((1, 600), lambda i: (0, 0)),
            pl.BlockSpec((600, EPAD), lambda i: (0, 0)),
            pl.BlockSpec((1, EPAD), lambda i: (0, 0)),
        ],
        out_specs=pl.BlockSpec((2, bn, HALF), lambda i: (0, i, 0)),
        out_shape=jax.ShapeDtypeStruct((2, NPAD, HALF), jnp.float32),
    )(scale, h2, agg2, w1, b1, w2, b2)


def _head_tc_body(h_ref, m_ref, w_ref, b_ref, o_ref):
    x = jnp.concatenate([h_ref[0, :, 0, :], h_ref[1, :, 0, :]], axis=-1)
    x = x * m_ref[...]
    o_ref[...] = jax.lax.dot_general(
        x, w_ref[...], (((1,), (0,)), ((), ())),
        preferred_element_type=jnp.float32) + b_ref[...]


def _head_tc(hr, mask_e, wp, bp):
    bn = 1000
    return pl.pallas_call(
        _head_tc_body,
        grid=(5000 // bn,),
        in_specs=[
            pl.BlockSpec((2, bn, 2, HALF), lambda i: (0, i, 0, 0)),
            pl.BlockSpec((bn, 1), lambda i: (i, 0)),
            pl.BlockSpec((EPAD, 128), lambda i: (0, 0)),
            pl.BlockSpec((1, 128), lambda i: (0, 0)),
        ],
        out_specs=pl.BlockSpec((bn, 128), lambda i: (i, 0)),
        out_shape=jax.ShapeDtypeStruct((5000, 128), jnp.float32),
    )(hr, mask_e, wp, bp)


@jax.jit
def kernel(x_atom, edge_index, edge_attr, node_mask, atom_emb, bond_emb, eps,
           W1, b1, W2, b2, pred_W, pred_b):
    N, NF = x_atom.shape
    E = edge_index.shape[1]
    NL = W1.shape[0]
    EMB = atom_emb.shape[-1]
    AD = atom_emb.shape[1]

    pad = EPAD - EMB
    ta = jnp.pad(atom_emb.reshape(NF * AD, EMB), ((0, 0), (0, pad)))
    ta = ta.reshape(NF * AD, 2, HALF).transpose(1, 0, 2).reshape(2 * NF * AD, HALF)
    d0, d1, d2 = BOND_DIMS
    bt = (bond_emb[:, 0, :d0, None, None, :]
          + bond_emb[:, 1, None, :d1, None, :]
          + bond_emb[:, 2, None, None, :d2, :]).reshape(NL, d0 * d1 * d2, EMB)
    bt = jnp.pad(bt, ((0, 0), (0, 0), (0, pad)))
    bt = bt.reshape(NL, d0 * d1 * d2, 2, HALF).transpose(0, 2, 1, 3)
    bt = bt.reshape(NL, 2 * d0 * d1 * d2, HALF)
    w1p = jnp.pad(W1, ((0, 0), (0, pad), (0, 0)))
    w2p = jnp.pad(W2, ((0, 0), (0, 0), (0, pad)))
    b1r = b1.reshape(NL, 1, 600)
    b2p = jnp.pad(b2, ((0, 0), (0, pad))).reshape(NL, 1, EPAD)
    wpp = jnp.pad(pred_W, ((0, pad), (0, 128 - pred_W.shape[1])))
    bpp = jnp.pad(pred_b, ((0, 128 - pred_b.shape[0]),)).reshape(1, 128)
    scales = (1.0 + eps).reshape(NL, 1, 1)

    aidx = (x_atom.astype(jnp.int32)
            + (jnp.arange(NF, dtype=jnp.int32) * AD)[None, :]).T
    aidx = jnp.pad(aidx, ((0, 0), (0, NPAD - N)))
    a4 = aidx.reshape(NF, NS, 5, CK).transpose(1, 2, 0, 3)
    a4 = a4.reshape(1, NS, 45, CK)
    aidx2 = jnp.concatenate([a4, a4 + NF * AD], axis=0)

    src = edge_index[0].astype(jnp.int32)
    dst = edge_index[1].astype(jnp.int32)
    combo = (edge_attr[:, 0].astype(jnp.int32) * (d1 * d2)
             + edge_attr[:, 1].astype(jnp.int32) * d2
             + edge_attr[:, 2].astype(jnp.int32))
    TPE = ((E // NS + CK - 1) // CK) * CK
    EP = TPE * NS
    src_p = jnp.pad(src, (0, EP - E)).reshape(1, NS, TPE // CK, CK)
    src2 = jnp.concatenate([src_p, src_p + NPAD], axis=0)
    dst_p = jnp.pad(dst, (0, EP - E), constant_values=NPAD)
    dst3 = dst_p.reshape(NS, TPE // CK, CK)
    com_p = jnp.pad(combo, (0, EP - E)).reshape(1, NS, TPE // CK, CK)
    combo2 = jnp.concatenate([com_p, com_p + d0 * d1 * d2], axis=0)
    mask_e = node_mask.reshape(N // 2, 2)[:, 0].astype(jnp.float32).reshape(N // 2, 1)

    h_flat = _atom_encoder(ta, aidx2)
    for l in range(NL):
        agg = _message_pass(h_flat, bt[l], src2, dst3, combo2)
        h2 = h_flat.reshape(2, NPAD, HALF)
        agg2 = agg.reshape(2, RPAD, HALF)
        h2 = _mlp_tc(h2, agg2, w1p[l], b1r[l], w2p[l], b2p[l],
                     scales[l], last=(l == NL - 1))
        h_flat = h2.reshape(2 * NPAD, HALF)
    hr = h2[:, :N, :].reshape(2, N // 2, 2, HALF)
    out = _head_tc(hr, mask_e, wpp, bpp)
    return out[:, :pred_W.shape[1]]


# speedup vs baseline: 3.0894x; 1.1008x over previous
import functools
import jax
import jax.numpy as jnp
from jax import lax
from jax.experimental import pallas as pl
from jax.experimental.pallas import tpu as pltpu
from jax.experimental.pallas import tpu_sc as plsc

NC, NS = 2, 16
EPAD, HALF = 320, 160
NVR = HALF // 16
NPAD = 10240
RPAD = 10048
CK = 128
ECK = 64
EB = 8
NEB = 20
BOND_DIMS = (5, 6, 2)


def _mesh():
    return plsc.VectorSubcoreMesh(core_axis_name="c", subcore_axis_name="s")


def _zero_rows(buf, nrows):
    z = jnp.zeros((16,), jnp.float32)
    def body(r, _):
        for k in range(NVR):
            buf[r, pl.ds(16 * k, 16)] = z
        return 0
    lax.fori_loop(0, nrows, body, 0)


def _atom_body(table_hbm, aidx_hbm, h_hbm, semA, semB, semC):
    c = lax.axis_index("c")
    s = lax.axis_index("s")
    def scoped(idxv, ga, gb, gc):
        _atom_inner(table_hbm, aidx_hbm, h_hbm, semA, semB, semC,
                    c, s, idxv, ga, gb, gc)
    pl.run_scoped(scoped,
                  pltpu.VMEM((45, CK), jnp.int32),
                  pltpu.VMEM((CK, HALF), jnp.float32),
                  pltpu.VMEM((CK, HALF), jnp.float32),
                  pltpu.VMEM((CK, HALF), jnp.float32))


def _atom_inner(table_hbm, aidx_hbm, h_hbm, semA, semB, semC,
                c, s, idxv, ga, gb, gc):
    pltpu.sync_copy(aidx_hbm.at[c].at[s], idxv)
    for j in range(5):
        bufs = ((ga, semA), (gb, semB), (gc, semC))
        d = [pltpu.async_copy(table_hbm.at[idxv.at[j * 9 + 3 * k]], buf, sem)
             for k, (buf, sem) in enumerate(bufs)]
        for step in (1, 2):
            for k, (buf, sem) in enumerate(bufs):
                d[k].wait()
                d[k] = pltpu.async_copy(
                    table_hbm.at[idxv.at[j * 9 + 3 * k + step]], buf, sem,
                    add=True)
        for k in range(3):
            d[k].wait()
        def row(r, _):
            for k in range(NVR):
                sl = pl.ds(16 * k, 16)
                ga[r, sl] = ga[r, sl] + gb[r, sl] + gc[r, sl]
            return 0
        lax.fori_loop(0, CK, row, 0)
        base = c * NPAD + s * 640 + j * CK
        pltpu.sync_copy(ga, h_hbm.at[pl.ds(base, CK)])


def _atom_encoder(table_flat, aidx2):
    k = pl.kernel(
        _atom_body,
        out_type=jax.ShapeDtypeStruct((NC * NPAD, HALF), jnp.float32),
        mesh=_mesh(),
        compiler_params=pltpu.CompilerParams(use_tc_tiling_on_sc=False),
        scratch_types=[
            pltpu.SemaphoreType.DMA,
            pltpu.SemaphoreType.DMA,
            pltpu.SemaphoreType.DMA,
        ],
    )
    return k(table_flat, aidx2)


def _layer_body(h_hbm, bond_hbm, eidx_hbm, agg_hbm, aggs,
                semg0, semg1, semb0, semb1, sems0, sems1):
    c = lax.axis_index("c")
    s = lax.axis_index("s")
    def scoped(idxv, g0, g1):
        _layer_inner(h_hbm, bond_hbm, eidx_hbm, agg_hbm, aggs,
                     (semg0, semg1), (semb0, semb1), (sems0, sems1),
                     c, s, idxv, (g0, g1))
    pl.run_scoped(scoped,
                  pltpu.VMEM((EB, 3, ECK), jnp.int32),
                  pltpu.VMEM((ECK, HALF), jnp.float32),
                  pltpu.VMEM((ECK, HALF), jnp.float32))


def _relu_rows(buf):
    def body(r, _):
        for k in range(NVR):
            sl = pl.ds(16 * k, 16)
            buf[r, sl] = jnp.maximum(buf[r, sl], 0.0)
        return 0
    lax.fori_loop(0, ECK, body, 0)


def _layer_inner(h_hbm, bond_hbm, eidx_hbm, agg_hbm, aggs,
                 semg, semb, sems, c, s, idxv, g):
    nz = RPAD // ECK
    _zero_rows(g[0], ECK)
    def zchunk(i, _):
        idx = s + NS * i
        @pl.when(idx < nz)
        def _():
            pltpu.sync_copy(g[0], aggs.at[pl.ds(idx * ECK, ECK)])
        return 0
    lax.fori_loop(0, (nz + NS - 1) // NS, zchunk, 0)
    plsc.subcore_barrier()

    def block(bi, _):
        pltpu.sync_copy(eidx_hbm.at[c].at[s].at[bi], idxv)
        dg = [None] * EB
        db = [None] * EB
        ds_ = [None] * EB
        dg[0] = pltpu.async_copy(h_hbm.at[idxv.at[0, 0]], g[0], semg[0])
        for k in range(EB):
            b = k % 2
            dg[k].wait()
            db[k] = pltpu.async_copy(bond_hbm.at[idxv.at[k, 1]], g[b],
                                     semb[b], add=True)
            if k + 1 < EB:
                if k >= 1:
                    ds_[k - 1].wait()
                dg[k + 1] = pltpu.async_copy(
                    h_hbm.at[idxv.at[k + 1, 0]], g[1 - b], semg[1 - b])
            db[k].wait()
            _relu_rows(g[b])
            ds_[k] = pltpu.async_copy(g[b], aggs.at[idxv.at[k, 2]],
                                      sems[b], add=True)
        ds_[EB - 2].wait()
        ds_[EB - 1].wait()
        return 0
    lax.fori_loop(0, NEB, block, 0)

    plsc.subcore_barrier()
    def ochunk(i, _):
        idx = s + NS * i
        @pl.when(idx < nz)
        def _():
            pltpu.sync_copy(aggs.at[pl.ds(idx * ECK, ECK)],
                            agg_hbm.at[pl.ds(c * RPAD + idx * ECK, ECK)])
        return 0
    lax.fori_loop(0, (nz + NS - 1) // NS, ochunk, 0)


def _message_pass(h_flat, bond_flat, eidx):
    k = pl.kernel(
        _layer_body,
        out_type=jax.ShapeDtypeStruct((NC * RPAD, HALF), jnp.float32),
        mesh=_mesh(),
        compiler_params=pltpu.CompilerParams(use_tc_tiling_on_sc=False),
        scratch_types=[
            pltpu.VMEM_SHARED((RPAD, HALF), jnp.float32),
            pltpu.SemaphoreType.DMA,
            pltpu.SemaphoreType.DMA,
            pltpu.SemaphoreType.DMA,
            pltpu.SemaphoreType.DMA,
            pltpu.SemaphoreType.DMA,
            pltpu.SemaphoreType.DMA,
        ],
    )
    return k(h_flat, bond_flat, eidx)


def _mlp_tc_body(scale_ref, h_ref, a_ref, w1_ref, b1_ref, w2_ref, b2_ref,
                 o_ref, *, last):
    x = jnp.concatenate([h_ref[0], h_ref[1]], axis=-1)
    a = jnp.concatenate([a_ref[0], a_ref[1]], axis=-1)
    z = x * scale_ref[0, 0] + a
    t = jnp.maximum(
        jax.lax.dot_general(z, w1_ref[...], (((1,), (0,)), ((), ())),
                            preferred_element_type=jnp.float32) + b1_ref[...],
        0.0)
    y = jax.lax.dot_general(t, w2_ref[...], (((1,), (0,)), ((), ())),
                            preferred_element_type=jnp.float32) + b2_ref[...]
    if not last:
        y = jnp.maximum(y, 0.0)
    o_ref[0] = y[:, :HALF]
    o_ref[1] = y[:, HALF:]


def _mlp_tc(h2, agg2, w1, b1, w2, b2, scale, last):
    bn = 1000
    grid = (10000 // bn,)
    return pl.pallas_call(
        functools.partial(_mlp_tc_body, last=last),
        grid=grid,
        in_specs=[
            pl.BlockSpec((1, 1), lambda i: (0, 0), memory_space=pltpu.SMEM),
            pl.BlockSpec((2, bn, HALF), lambda i: (0, i, 0)),
            pl.BlockSpec((2, bn, HALF), lambda i: (0, i, 0)),
            pl.BlockSpec((EPAD, 600), lambda i: (0, 0)),
            pl.BlockSpec((1, 600), lambda i: (0, 0)),
            pl.BlockSpec((600, EPAD), lambda i: (0, 0)),
            pl.BlockSpec((1, EPAD), lambda i: (0, 0)),
        ],
        out_specs=pl.BlockSpec((2, bn, HALF), lambda i: (0, i, 0)),
        out_shape=jax.ShapeDtypeStruct((2, NPAD, HALF), jnp.float32),
    )(scale, h2, agg2, w1, b1, w2, b2)


def _head_tc_body(h_ref, m_ref, w_ref, b_ref, o_ref):
    x = jnp.concatenate([h_ref[0, :, 0, :], h_ref[1, :, 0, :]], axis=-1)
    x = x * m_ref[...]
    o_ref[...] = jax.lax.dot_general(
        x, w_ref[...], (((1,), (0,)), ((), ())),
        preferred_element_type=jnp.float32) + b_ref[...]


def _head_tc(hr, mask_e, wp, bp):
    bn = 1000
    return pl.pallas_call(
        _head_tc_body,
        grid=(5000 // bn,),
        in_specs=[
            pl.BlockSpec((2, bn, 2, HALF), lambda i: (0, i, 0, 0)),
            pl.BlockSpec((bn, 1), lambda i: (i, 0)),
            pl.BlockSpec((EPAD, 128), lambda i: (0, 0)),
            pl.BlockSpec((1, 128), lambda i: (0, 0)),
        ],
        out_specs=pl.BlockSpec((bn, 128), lambda i: (i, 0)),
        out_shape=jax.ShapeDtypeStruct((5000, 128), jnp.float32),
    )(hr, mask_e, wp, bp)


@jax.jit
def kernel(x_atom, edge_index, edge_attr, node_mask, atom_emb, bond_emb, eps,
           W1, b1, W2, b2, pred_W, pred_b):
    N, NF = x_atom.shape
    E = edge_index.shape[1]
    NL = W1.shape[0]
    EMB = atom_emb.shape[-1]
    AD = atom_emb.shape[1]

    pad = EPAD - EMB
    ta = jnp.pad(atom_emb.reshape(NF * AD, EMB), ((0, 0), (0, pad)))
    ta = ta.reshape(NF * AD, 2, HALF).transpose(1, 0, 2).reshape(2 * NF * AD, HALF)
    d0, d1, d2 = BOND_DIMS
    nb = d0 * d1 * d2
    bt = (bond_emb[:, 0, :d0, None, None, :]
          + bond_emb[:, 1, None, :d1, None, :]
          + bond_emb[:, 2, None, None, :d2, :]).reshape(NL, nb, EMB)
    bt = jnp.pad(bt, ((0, 0), (0, 0), (0, pad)))
    bt = bt.reshape(NL, nb, 2, HALF).transpose(0, 2, 1, 3).reshape(NL, 2 * nb, HALF)
    w1p = jnp.pad(W1, ((0, 0), (0, pad), (0, 0)))
    w2p = jnp.pad(W2, ((0, 0), (0, 0), (0, pad)))
    b1r = b1.reshape(NL, 1, 600)
    b2p = jnp.pad(b2, ((0, 0), (0, pad))).reshape(NL, 1, EPAD)
    wpp = jnp.pad(pred_W, ((0, pad), (0, 128 - pred_W.shape[1])))
    bpp = jnp.pad(pred_b, ((0, 128 - pred_b.shape[0]),)).reshape(1, 128)
    scales = (1.0 + eps).reshape(NL, 1, 1)

    aidx = (x_atom.astype(jnp.int32)
            + (jnp.arange(NF, dtype=jnp.int32) * AD)[None, :]).T
    aidx = jnp.pad(aidx, ((0, 0), (0, NPAD - N)))
    a4 = aidx.reshape(NF, NS, 5, CK).transpose(1, 2, 0, 3)
    a4 = a4.reshape(1, NS, 45, CK)
    aidx2 = jnp.concatenate([a4, a4 + NF * AD], axis=0)

    src = edge_index[0].astype(jnp.int32)
    dst = edge_index[1].astype(jnp.int32)
    combo = (edge_attr[:, 0].astype(jnp.int32) * (d1 * d2)
             + edge_attr[:, 1].astype(jnp.int32) * d2
             + edge_attr[:, 2].astype(jnp.int32))
    TPE = NEB * EB * ECK
    EP = TPE * NS
    src_p = jnp.pad(src, (0, EP - E)).reshape(NS, NEB, EB, ECK)
    dst_p = jnp.pad(dst, (0, EP - E), constant_values=10000)
    dst_p = dst_p.reshape(NS, NEB, EB, ECK)
    com_p = jnp.pad(combo, (0, EP - E)).reshape(NS, NEB, EB, ECK)
    e0 = jnp.stack([src_p, com_p, dst_p], axis=3)
    e1 = jnp.stack([src_p + NPAD, com_p + nb, dst_p], axis=3)
    eidx = jnp.stack([e0, e1], axis=0)
    mask_e = node_mask.reshape(N // 2, 2)[:, 0].astype(jnp.float32).reshape(N // 2, 1)

    h_flat = _atom_encoder(ta, aidx2)
    for l in range(NL):
        agg = _message_pass(h_flat, bt[l], eidx)
        h2 = h_flat.reshape(2, NPAD, HALF)
        agg2 = agg.reshape(2, RPAD, HALF)
        h2 = _mlp_tc(h2, agg2, w1p[l], b1r[l], w2p[l], b2p[l],
                     scales[l], last=(l == NL - 1))
        h_flat = h2.reshape(2 * NPAD, HALF)
    hr = h2[:, :N, :].reshape(2, N // 2, 2, HALF)
    out = _head_tc(hr, mask_e, wpp, bpp)
    return out[:, :pred_W.shape[1]]
